# baseline (speedup 1.0000x reference)
import functools

import jax
import jax.numpy as jnp
from jax import lax
from jax.experimental import pallas as pl
from jax.experimental.pallas import tpu as pltpu
from jax.experimental.pallas import tpu_sc as plsc

VOCAB = 1000000
DIM = 32
BATCH = 4096
HIST = 50
B = BATCH * HIST

NC = 2
NS = 16
NW = NC * NS
BPW = B // NW
C = 1280
NCHUNK = BPW // C

TBLK = 24576
QBLKS = (VOCAB + 4 * TBLK - 1) // (4 * TBLK)
QPAD = QBLKS * TBLK
SRC_LAST_BLK = (VOCAB - 1) // TBLK


def _tp_body(q0, q1, q2, q3, o_ref):
    x = jnp.concatenate([q0[...], q1[...], q2[...], q3[...]], axis=0)
    o_ref[...] = x.T


def _q_spec(a):
    return pl.BlockSpec(
        (DIM, TBLK),
        lambda i, a=a: (0, jnp.minimum(a * QBLKS + i, SRC_LAST_BLK)),
    )


_transpose = pl.pallas_call(
    _tp_body,
    grid=(QBLKS,),
    in_specs=[_q_spec(a) for a in range(4)],
    out_specs=pl.BlockSpec((TBLK, 128), lambda i: (i, 0)),
    out_shape=jax.ShapeDtypeStruct((QPAD, 128), jnp.float32),
)

OBLK = 512


def _tpo_body(x_ref, o_ref):
    o_ref[...] = x_ref[...].T


_transpose_out = pl.pallas_call(
    _tpo_body,
    grid=(BATCH // OBLK,),
    in_specs=[pl.BlockSpec((OBLK, HIST * DIM), lambda i: (i, 0))],
    out_specs=pl.BlockSpec((HIST * DIM, OBLK), lambda i: (0, i)),
    out_shape=jax.ShapeDtypeStruct((HIST * DIM, BATCH), jnp.float32),
)

VROWS = 4 * QPAD

_mesh = plsc.VectorSubcoreMesh(core_axis_name="c", subcore_axis_name="s")


@functools.partial(
    pl.kernel,
    out_type=jax.ShapeDtypeStruct((B, DIM), jnp.float32),
    mesh=_mesh,
    scratch_types=[
        pltpu.VMEM((BPW,), jnp.int32),
        pltpu.VMEM((2, C, DIM), jnp.float32),
        pltpu.SemaphoreType.DMA,
        pltpu.SemaphoreType.DMA,
    ],
    compiler_params=pltpu.CompilerParams(use_tc_tiling_on_sc=False),
)
def _sc_gather(idx_hbm, table_hbm, out_hbm, idx_v, rows_v, sem0, sem1):
    wid = lax.axis_index("s") * NC + lax.axis_index("c")
    base = wid * BPW
    pltpu.sync_copy(idx_hbm.at[pl.ds(base, BPW)], idx_v)

    sems = (sem0, sem1)

    def issue(j, slot):
        return pltpu.async_copy(
            table_hbm.at[idx_v.at[pl.ds(j * C, C)]], rows_v.at[slot], sems[slot]
        )

    issue(0, 0)
    for j in range(NCHUNK):
        slot = j % 2
        if j + 1 < NCHUNK:
            issue(j + 1, 1 - slot)
        pltpu.make_async_copy(
            table_hbm.at[idx_v.at[pl.ds(j * C, C)]], rows_v.at[slot], sems[slot]
        ).wait()
        pltpu.sync_copy(rows_v.at[slot], out_hbm.at[pl.ds(base + j * C, C)])


def kernel(inputs, table):
    idx_v = (inputs % QPAD) * 4 + inputs // QPAD
    table_t = table.T
    packed = _transpose(table_t, table_t, table_t, table_t)
    table_view = packed.reshape(VROWS, DIM)
    out = _sc_gather(idx_v.reshape(B), table_view)
    out2 = _transpose_out(out.reshape(BATCH, HIST * DIM))
    return out2.reshape(HIST, DIM, BATCH).transpose(2, 0, 1)


# speedup vs baseline: 1.0082x; 1.0082x over previous
import functools

import jax
import jax.numpy as jnp
from jax import lax
from jax.experimental import pallas as pl
from jax.experimental.pallas import tpu as pltpu
from jax.experimental.pallas import tpu_sc as plsc

VOCAB = 1000000
DIM = 32
BATCH = 4096
HIST = 50
B = BATCH * HIST

NC = 2
NS = 16
NW = NC * NS
BPW = B // NW
C = 1280
NCHUNK = BPW // C

TBLK = 16384
QBLKS = (VOCAB + 4 * TBLK - 1) // (4 * TBLK)
QPAD = QBLKS * TBLK
SRC_LAST_BLK = (VOCAB - 1) // TBLK


def _tp_body(q0, q1, q2, q3, o_ref):
    x = jnp.concatenate([q0[...], q1[...], q2[...], q3[...]], axis=0)
    o_ref[...] = x.T


def _q_spec(a):
    return pl.BlockSpec(
        (DIM, TBLK),
        lambda i, a=a: (0, jnp.minimum(a * QBLKS + i, SRC_LAST_BLK)),
    )


_transpose = pl.pallas_call(
    _tp_body,
    grid=(QBLKS,),
    in_specs=[_q_spec(a) for a in range(4)],
    out_specs=pl.BlockSpec((TBLK, 128), lambda i: (i, 0)),
    out_shape=jax.ShapeDtypeStruct((QPAD, 128), jnp.float32),
)

OBLK = 1024


def _tpo_body(x_ref, o_ref):
    o_ref[...] = x_ref[...].T


_transpose_out = pl.pallas_call(
    _tpo_body,
    grid=(BATCH // OBLK,),
    in_specs=[pl.BlockSpec((OBLK, HIST * DIM), lambda i: (i, 0))],
    out_specs=pl.BlockSpec((HIST * DIM, OBLK), lambda i: (0, i)),
    out_shape=jax.ShapeDtypeStruct((HIST * DIM, BATCH), jnp.float32),
)

VROWS = 4 * QPAD

_mesh = plsc.VectorSubcoreMesh(core_axis_name="c", subcore_axis_name="s")


@functools.partial(
    pl.kernel,
    out_type=jax.ShapeDtypeStruct((B, DIM), jnp.float32),
    mesh=_mesh,
    scratch_types=[
        pltpu.VMEM((BPW,), jnp.int32),
        pltpu.VMEM((2, C, DIM), jnp.float32),
        pltpu.SemaphoreType.DMA,
        pltpu.SemaphoreType.DMA,
    ],
    compiler_params=pltpu.CompilerParams(use_tc_tiling_on_sc=False),
)
def _sc_gather(idx_hbm, table_hbm, out_hbm, idx_v, rows_v, sem0, sem1):
    wid = lax.axis_index("s") * NC + lax.axis_index("c")
    base = wid * BPW
    pltpu.sync_copy(idx_hbm.at[pl.ds(base, BPW)], idx_v)

    sems = (sem0, sem1)

    def issue(j, slot):
        return pltpu.async_copy(
            table_hbm.at[idx_v.at[pl.ds(j * C, C)]], rows_v.at[slot], sems[slot]
        )

    issue(0, 0)
    for j in range(NCHUNK):
        slot = j % 2
        if j + 1 < NCHUNK:
            issue(j + 1, 1 - slot)
        pltpu.make_async_copy(
            table_hbm.at[idx_v.at[pl.ds(j * C, C)]], rows_v.at[slot], sems[slot]
        ).wait()
        pltpu.sync_copy(rows_v.at[slot], out_hbm.at[pl.ds(base + j * C, C)])


def kernel(inputs, table):
    idx_v = (inputs % QPAD) * 4 + inputs // QPAD
    table_t = table.T
    packed = _transpose(table_t, table_t, table_t, table_t)
    table_view = packed.reshape(VROWS, DIM)
    out = _sc_gather(idx_v.reshape(B), table_view)
    out2 = _transpose_out(out.reshape(BATCH, HIST * DIM))
    return out2.reshape(HIST, DIM, BATCH).transpose(2, 0, 1)
